# baseline (speedup 1.0000x reference)
import functools

import jax
import jax.numpy as jnp
from jax import lax
from jax.experimental import pallas as pl
from jax.experimental.pallas import tpu as pltpu
from jax.experimental.pallas import tpu_sc as plsc

_NUM_CORES = 2
_NUM_SUBCORES = 16
_NUM_WORKERS = _NUM_CORES * _NUM_SUBCORES


@functools.partial(jax.jit, static_argnums=(2, 3))
def _embedding_gather(idx, weight, B, D):
    b_per_w = B // _NUM_WORKERS
    C = 1280
    nchunks = b_per_w // C
    mesh = plsc.VectorSubcoreMesh(core_axis_name="c", subcore_axis_name="s")

    @functools.partial(
        pl.kernel,
        mesh=mesh,
        out_type=jax.ShapeDtypeStruct((B, D), jnp.float32),
        scratch_types=[
            pltpu.VMEM((2, C), jnp.int32),
            pltpu.VMEM((2, C, D), jnp.float32),
            pltpu.SemaphoreType.DMA((2,)),
            pltpu.SemaphoreType.DMA((2,)),
            pltpu.SemaphoreType.DMA((2,)),
        ],
        compiler_params=pltpu.CompilerParams(use_tc_tiling_on_sc=False),
    )
    def k(idx_hbm, table_hbm, out_hbm, idx_v, rows_v, sem_i, sem_g, sem_s):
        wid = lax.axis_index("s") * _NUM_CORES + lax.axis_index("c")
        base = wid * b_per_w

        idx_cp = [None] * nchunks
        gat_cp = [None] * nchunks
        st_cp = [None] * nchunks

        def start_idx(i):
            b = i % 2
            idx_cp[i] = pltpu.make_async_copy(
                idx_hbm.at[pl.ds(base + i * C, C)], idx_v.at[b], sem_i.at[b]
            )
            idx_cp[i].start()

        start_idx(0)
        for i in range(nchunks):
            b = i % 2
            if i + 1 < nchunks:
                start_idx(i + 1)
            idx_cp[i].wait()
            if i >= 2:
                st_cp[i - 2].wait()
            gat_cp[i] = pltpu.make_async_copy(
                table_hbm.at[idx_v.at[b]], rows_v.at[b], sem_g.at[b]
            )
            gat_cp[i].start()
            gat_cp[i].wait()
            st_cp[i] = pltpu.make_async_copy(
                rows_v.at[b], out_hbm.at[pl.ds(base + i * C, C)], sem_s.at[b]
            )
            st_cp[i].start()
        st_cp[nchunks - 2].wait()
        st_cp[nchunks - 1].wait()

    return k(idx, weight)


def kernel(input, weight):
    N, W = input.shape
    D = weight.shape[1]
    B = N * W
    idx = input.T.reshape(B).astype(jnp.int32)
    out = _embedding_gather(idx, weight, B, D)
    return out.reshape(W, N, D).transpose(1, 0, 2)
